# baseline (speedup 1.0000x reference)
import functools

import jax
import jax.numpy as jnp
from jax import lax
from jax.experimental import pallas as pl
from jax.experimental.pallas import tpu as pltpu
from jax.experimental.pallas import tpu_sc as plsc

B = 16384
D = 64
CH = 128
GRP = 16


def _rsqrt16(x):
    xi = plsc.bitcast(x, jnp.int32)
    yi = jnp.int32(0x5F3759DF) - lax.shift_right_logical(xi, 1)
    y = plsc.bitcast(yi, jnp.float32)
    for _ in range(3):
        y = y * (1.5 - 0.5 * x * y * y)
    return y


@functools.cache
def _make_sc_kernel():
    info = plsc.get_sparse_core_info()
    nw = info.num_cores * info.num_subcores
    b_per_w = B // nw
    n_ch = b_per_w // CH
    mesh = plsc.VectorSubcoreMesh(core_axis_name="c", subcore_axis_name="s")

    @functools.partial(
        pl.kernel,
        mesh=mesh,
        out_type=jax.ShapeDtypeStruct((D, B), jnp.float32),
        compiler_params=pltpu.CompilerParams(
            use_tc_tiling_on_sc=False, needs_layout_passes=False
        ),
        scratch_types=[
            pltpu.VMEM((n_ch, CH), jnp.int32),
            pltpu.VMEM((b_per_w, D), jnp.float32),
            pltpu.VMEM((D, b_per_w), jnp.float32),
            pltpu.VMEM((b_per_w,), jnp.float32),
            pltpu.VMEM((b_per_w,), jnp.float32),
            pltpu.VMEM((b_per_w,), jnp.float32),
            pltpu.VMEM((D, GRP), jnp.float32),
            pltpu.VMEM((D, GRP), jnp.float32),
            pltpu.VMEM((D, GRP), jnp.float32),
            pltpu.VMEM((D, GRP), jnp.float32),
            pltpu.VMEM((D, GRP), jnp.float32),
            pltpu.SemaphoreType.DMA,
            pltpu.SemaphoreType.DMA,
        ],
    )
    def sc_kernel(idx_hbm, table_hbm, c0_hbm, c1_hbm, m_hbm, wrep_hbm,
                  brep_hbm, out_hbm, idx_v, rows_v, col_v, c0_v, c1_v, m_v,
                  w0_v, w1_v, b_v, spa_col, g_col, gsem, wsem):
        wid = lax.axis_index("s") * info.num_cores + lax.axis_index("c")
        base = wid * b_per_w

        pltpu.sync_copy(idx_hbm.at[pl.ds(wid * n_ch, n_ch)], idx_v)
        copies = [
            pltpu.async_copy(
                table_hbm.at[idx_v.at[j]], rows_v.at[pl.ds(j * CH, CH)], gsem
            )
            for j in range(n_ch)
        ]
        pltpu.sync_copy(c0_hbm.at[pl.ds(base, b_per_w)], c0_v)
        pltpu.sync_copy(c1_hbm.at[pl.ds(base, b_per_w)], c1_v)
        pltpu.sync_copy(m_hbm.at[pl.ds(base, b_per_w)], m_v)
        pltpu.sync_copy(wrep_hbm.at[0], w0_v)
        pltpu.sync_copy(wrep_hbm.at[1], w1_v)
        pltpu.sync_copy(brep_hbm, b_v)

        iota = lax.iota(jnp.int32, GRP)
        zero16 = jnp.zeros((GRP,), jnp.float32)

        def group_body(r0):
            rowvec = r0 + iota
            c0g = c0_v[pl.ds(r0, GRP)]
            c1g = c1_v[pl.ds(r0, GRP)]
            mg = m_v[pl.ds(r0, GRP)]
            accg = zero16
            accs = zero16
            for d in range(D):
                dvec = jnp.full((GRP,), d, jnp.int32)
                gd = plsc.load_gather(rows_v, [rowvec, dvec])
                accg = accg + gd * gd
                spad = c0g * w0_v[d] + c1g * w1_v[d] + b_v[d]
                accs = accs + spad * spad
                spa_col[d] = spad
                g_col[d] = gd
            rg = _rsqrt16(accg)
            sv = (1.0 - mg) * accs + mg * (accg * rg * rg)
            fs = _rsqrt16(sv)
            a = (1.0 - mg) * fs
            bco = mg * (rg * fs)
            for d in range(D):
                col_v[d, pl.ds(r0, GRP)] = spa_col[d] * a + g_col[d] * bco

        for j in range(n_ch):
            copies[j].wait()

            def chunk_loop(k, carry):
                group_body(j * CH + k * GRP)
                return carry

            lax.fori_loop(0, CH // GRP, chunk_loop, 0)

        writes = [
            pltpu.async_copy(col_v.at[d], out_hbm.at[d, pl.ds(base, b_per_w)],
                             wsem)
            for d in range(D)
        ]
        for w in writes:
            w.wait()

    return sc_kernel


def kernel(coords, nogeo_khot, nogeo_ids, W_spa, b_spa, nogeo_table):
    idx2 = nogeo_ids.reshape(B // CH, CH)
    c0 = coords[:, 0, 0]
    c1 = coords[:, 0, 1]
    mf = nogeo_khot.astype(jnp.float32)
    wrep = jnp.broadcast_to(W_spa[:, :, None], (2, D, GRP))
    brep = jnp.broadcast_to(b_spa[:, None], (D, GRP))
    return _make_sc_kernel()(idx2, nogeo_table, c0, c1, mf, wrep, brep)


# speedup vs baseline: 1.0049x; 1.0049x over previous
import functools

import jax
import jax.numpy as jnp
from jax import lax
from jax.experimental import pallas as pl
from jax.experimental.pallas import tpu as pltpu
from jax.experimental.pallas import tpu_sc as plsc

B = 16384
D = 64
CH = 128
GRP = 16


def _rsqrt16(x):
    xi = plsc.bitcast(x, jnp.int32)
    yi = jnp.int32(0x5F3759DF) - lax.shift_right_logical(xi, 1)
    y = plsc.bitcast(yi, jnp.float32)
    for _ in range(3):
        y = y * (1.5 - 0.5 * x * y * y)
    return y


@functools.cache
def _make_sc_kernel():
    info = plsc.get_sparse_core_info()
    nw = info.num_cores * info.num_subcores
    b_per_w = B // nw
    n_ch = b_per_w // CH
    n_grp = CH // GRP
    mesh = plsc.VectorSubcoreMesh(core_axis_name="c", subcore_axis_name="s")

    @functools.partial(
        pl.kernel,
        mesh=mesh,
        out_type=jax.ShapeDtypeStruct((D, B), jnp.float32),
        compiler_params=pltpu.CompilerParams(
            use_tc_tiling_on_sc=False, needs_layout_passes=False
        ),
        scratch_types=[
            pltpu.VMEM((n_ch, CH), jnp.int32),
            pltpu.VMEM((b_per_w, D), jnp.float32),
            pltpu.VMEM((D, b_per_w), jnp.float32),
            pltpu.VMEM((b_per_w, 2), jnp.float32),
            pltpu.VMEM((b_per_w,), jnp.int32),
            pltpu.VMEM((D, GRP), jnp.float32),
            pltpu.VMEM((D, GRP), jnp.float32),
            pltpu.VMEM((D, GRP), jnp.float32),
            pltpu.VMEM((D, GRP), jnp.float32),
        ]
        + [pltpu.SemaphoreType.DMA] * (n_ch + 1),
    )
    def sc_kernel(idx_hbm, table_hbm, c01_hbm, m_hbm, wrep_hbm, brep_hbm,
                  out_hbm, idx_v, rows_v, col_v, c01_v, mi_v, w0_v, w1_v,
                  b_v, spa_col, *sems):
        gsems = sems[:n_ch]
        wsem = sems[n_ch]
        wid = lax.axis_index("s") * info.num_cores + lax.axis_index("c")
        base = wid * b_per_w

        for j in range(n_ch):
            pltpu.sync_copy(idx_hbm.at[pl.ds(base + j * CH, CH)], idx_v.at[j])
        copies = [None] * n_ch
        copies[0] = pltpu.async_copy(
            table_hbm.at[idx_v.at[0]], rows_v.at[pl.ds(0, CH)], gsems[0]
        )

        pltpu.sync_copy(c01_hbm.at[pl.ds(base, b_per_w)], c01_v)
        pltpu.sync_copy(m_hbm.at[pl.ds(base, b_per_w)], mi_v)
        pltpu.sync_copy(wrep_hbm.at[0], w0_v)
        pltpu.sync_copy(wrep_hbm.at[1], w1_v)
        pltpu.sync_copy(brep_hbm, b_v)

        iota = lax.iota(jnp.int32, GRP)
        z16 = jnp.zeros((GRP,), jnp.int32)
        o16 = jnp.ones((GRP,), jnp.int32)
        dvecs = [jnp.full((GRP,), d, jnp.int32) for d in range(D)]

        def phase_s(k, carry):
            r0 = k * GRP
            rowvec = r0 + iota
            c0g = plsc.load_gather(c01_v, [rowvec, z16])
            c1g = plsc.load_gather(c01_v, [rowvec, o16])
            mgf = mi_v[pl.ds(r0, GRP)].astype(jnp.float32)
            acc0 = jnp.zeros((GRP,), jnp.float32)
            acc1 = jnp.zeros((GRP,), jnp.float32)
            for d in range(D):
                w0 = w0_v[d]
                w1 = w1_v[d]
                bb = b_v[d]
                spad = c0g * w0 + c1g * w1 + bb
                if d % 2 == 0:
                    acc0 = acc0 + spad * spad
                else:
                    acc1 = acc1 + spad * spad
                spa_col[d] = spad
            ag = (1.0 - mgf) * _rsqrt16(acc0 + acc1)
            for d in range(D):
                col_v[d, pl.ds(r0, GRP)] = spa_col[d] * ag
            return carry

        lax.fori_loop(0, b_per_w // GRP, phase_s, 0)

        for j in range(n_ch):
            if j + 1 < n_ch:
                copies[j + 1] = pltpu.async_copy(
                    table_hbm.at[idx_v.at[j + 1]],
                    rows_v.at[pl.ds((j + 1) * CH, CH)],
                    gsems[j + 1],
                )
            copies[j].wait()

            def phase_g(k, carry, j=j):
                r0 = j * CH + k * GRP
                rowvec = r0 + iota
                mbool = mi_v[pl.ds(r0, GRP)] == 1
                acc0 = jnp.zeros((GRP,), jnp.float32)
                acc1 = jnp.zeros((GRP,), jnp.float32)
                for d in range(D):
                    gd = plsc.load_gather(rows_v, [rowvec, dvecs[d]])
                    if d % 2 == 0:
                        acc0 = acc0 + gd * gd
                    else:
                        acc1 = acc1 + gd * gd
                accg = acc0 + acc1
                rg = _rsqrt16(accg)
                fs = _rsqrt16(accg * rg * rg)
                bco = rg * fs
                for d in range(D):
                    gd = plsc.load_gather(rows_v, [rowvec, dvecs[d]])
                    cur = col_v[d, pl.ds(r0, GRP)]
                    col_v[d, pl.ds(r0, GRP)] = jnp.where(mbool, gd * bco, cur)
                return carry

            lax.fori_loop(0, n_grp, phase_g, 0)

        writes = [
            pltpu.async_copy(col_v.at[d], out_hbm.at[d, pl.ds(base, b_per_w)],
                             wsem)
            for d in range(D)
        ]
        for w in writes:
            w.wait()

    return sc_kernel


def kernel(coords, nogeo_khot, nogeo_ids, W_spa, b_spa, nogeo_table):
    c01 = coords.reshape(B, 2)
    wrep = jnp.broadcast_to(W_spa[:, :, None], (2, D, GRP))
    brep = jnp.broadcast_to(b_spa[:, None], (D, GRP))
    return _make_sc_kernel()(
        nogeo_ids, nogeo_table, c01, nogeo_khot, wrep, brep
    )


# speedup vs baseline: 1.0414x; 1.0362x over previous
import functools

import jax
import jax.numpy as jnp
from jax import lax
from jax.experimental import pallas as pl
from jax.experimental.pallas import tpu as pltpu
from jax.experimental.pallas import tpu_sc as plsc

B = 16384
D = 64
CH = 128
GRP = 16


def _rsqrt16(x):
    xi = plsc.bitcast(x, jnp.int32)
    yi = jnp.int32(0x5F3759DF) - lax.shift_right_logical(xi, 1)
    y = plsc.bitcast(yi, jnp.float32)
    for _ in range(3):
        y = y * (1.5 - 0.5 * x * y * y)
    return y


@functools.cache
def _make_sc_kernel():
    info = plsc.get_sparse_core_info()
    nw = info.num_cores * info.num_subcores
    b_per_w = B // nw
    n_ch = b_per_w // CH
    n_grp = CH // GRP
    mesh = plsc.VectorSubcoreMesh(core_axis_name="c", subcore_axis_name="s")

    @functools.partial(
        pl.kernel,
        mesh=mesh,
        out_type=jax.ShapeDtypeStruct((D // 8, B // 128, 8, 128), jnp.float32),
        compiler_params=pltpu.CompilerParams(
            use_tc_tiling_on_sc=False, needs_layout_passes=False
        ),
        scratch_types=[
            pltpu.VMEM((n_ch, CH), jnp.int32),
            pltpu.VMEM((b_per_w, D), jnp.float32),
            pltpu.VMEM((D, b_per_w), jnp.float32),
            pltpu.VMEM((b_per_w, 2), jnp.float32),
            pltpu.VMEM((b_per_w,), jnp.int32),
            pltpu.VMEM((D, GRP), jnp.float32),
            pltpu.VMEM((D, GRP), jnp.float32),
            pltpu.VMEM((D, GRP), jnp.float32),
            pltpu.VMEM((D, GRP), jnp.float32),
        ]
        + [pltpu.SemaphoreType.DMA] * (n_ch + 1),
    )
    def sc_kernel(idx_hbm, table_hbm, c01_hbm, m_hbm, wrep_hbm, brep_hbm,
                  out_hbm, idx_v, rows_v, col_v, c01_v, mi_v, w0_v, w1_v,
                  b_v, spa_col, *sems):
        gsems = sems[:n_ch]
        wsem = sems[n_ch]
        wid = lax.axis_index("s") * info.num_cores + lax.axis_index("c")
        base = wid * b_per_w

        for j in range(n_ch):
            pltpu.sync_copy(idx_hbm.at[pl.ds(base + j * CH, CH)], idx_v.at[j])
        copies = [None] * n_ch
        copies[0] = pltpu.async_copy(
            table_hbm.at[idx_v.at[0]], rows_v.at[pl.ds(0, CH)], gsems[0]
        )

        pltpu.sync_copy(c01_hbm.at[pl.ds(base, b_per_w)], c01_v)
        pltpu.sync_copy(m_hbm.at[pl.ds(base, b_per_w)], mi_v)
        pltpu.sync_copy(wrep_hbm.at[0], w0_v)
        pltpu.sync_copy(wrep_hbm.at[1], w1_v)
        pltpu.sync_copy(brep_hbm, b_v)

        iota = lax.iota(jnp.int32, GRP)
        z16 = jnp.zeros((GRP,), jnp.int32)
        o16 = jnp.ones((GRP,), jnp.int32)
        dvecs = [jnp.full((GRP,), d, jnp.int32) for d in range(D)]

        def phase_s(k, carry):
            r0 = k * GRP
            rowvec = r0 + iota
            c0g = plsc.load_gather(c01_v, [rowvec, z16])
            c1g = plsc.load_gather(c01_v, [rowvec, o16])
            mgf = mi_v[pl.ds(r0, GRP)].astype(jnp.float32)
            acc0 = jnp.zeros((GRP,), jnp.float32)
            acc1 = jnp.zeros((GRP,), jnp.float32)
            for d in range(D):
                w0 = w0_v[d]
                w1 = w1_v[d]
                bb = b_v[d]
                spad = c0g * w0 + c1g * w1 + bb
                if d % 2 == 0:
                    acc0 = acc0 + spad * spad
                else:
                    acc1 = acc1 + spad * spad
                spa_col[d] = spad
            ag = (1.0 - mgf) * _rsqrt16(acc0 + acc1)
            for d in range(D):
                col_v[d, pl.ds(r0, GRP)] = spa_col[d] * ag
            return carry

        lax.fori_loop(0, b_per_w // GRP, phase_s, 0)

        for j in range(n_ch):
            if j + 1 < n_ch:
                copies[j + 1] = pltpu.async_copy(
                    table_hbm.at[idx_v.at[j + 1]],
                    rows_v.at[pl.ds((j + 1) * CH, CH)],
                    gsems[j + 1],
                )
            copies[j].wait()

            def phase_g(k, carry, j=j):
                r0 = j * CH + k * GRP
                rowvec = r0 + iota
                mbool = mi_v[pl.ds(r0, GRP)] == 1
                acc0 = jnp.zeros((GRP,), jnp.float32)
                acc1 = jnp.zeros((GRP,), jnp.float32)
                for d in range(D):
                    gd = plsc.load_gather(rows_v, [rowvec, dvecs[d]])
                    if d % 2 == 0:
                        acc0 = acc0 + gd * gd
                    else:
                        acc1 = acc1 + gd * gd
                accg = acc0 + acc1
                rg = _rsqrt16(accg)
                fs = _rsqrt16(accg * rg * rg)
                bco = rg * fs
                for d in range(D):
                    gd = plsc.load_gather(rows_v, [rowvec, dvecs[d]])
                    cur = col_v[d, pl.ds(r0, GRP)]
                    col_v[d, pl.ds(r0, GRP)] = jnp.where(mbool, gd * bco, cur)
                return carry

            lax.fori_loop(0, n_grp, phase_g, 0)

        jbase = base // 128
        writes = [
            pltpu.async_copy(
                col_v.at[pl.ds(8 * i, 8), pl.ds(128 * j, 128)],
                out_hbm.at[i, jbase + j],
                wsem,
            )
            for i in range(D // 8)
            for j in range(b_per_w // 128)
        ]
        for w in writes:
            w.wait()

    return sc_kernel


def kernel(coords, nogeo_khot, nogeo_ids, W_spa, b_spa, nogeo_table):
    c01 = coords.reshape(B, 2)
    wrep = jnp.broadcast_to(W_spa[:, :, None], (2, D, GRP))
    brep = jnp.broadcast_to(b_spa[:, None], (D, GRP))
    out4 = _make_sc_kernel()(
        nogeo_ids, nogeo_table, c01, nogeo_khot, wrep, brep
    )
    return jnp.transpose(out4, (0, 2, 1, 3)).reshape(D, B)
